# baseline (speedup 1.0000x reference)
import functools

import jax
import jax.numpy as jnp
from jax import lax
from jax.experimental import pallas as pl
from jax.experimental.pallas import tpu as pltpu
from jax.experimental.pallas import tpu_sc as plsc

VOCAB = 1000000
EMBED_DIM = 64
MAX_SEQ_LEN = 200
SEQ_LEN = 200
BATCH = 4096

NC = 2
NS = 16
NW = NC * NS
L = 16

TOTAL_ROWS = BATCH * SEQ_LEN
ROWS_PER_W = TOTAL_ROWS // NW
CHUNK = 512
N_CHUNKS = ROWS_PER_W // CHUNK
IDX_W = 128
GATHERS = CHUNK // IDX_W

_EPS = 1e-5
_INV_D = 1.0 / EMBED_DIM


_GDN = lax.GatherDimensionNumbers(
    offset_dims=(), collapsed_slice_dims=(0,), start_index_map=(0,))


def _permute(v, perm):
    return lax.gather(v, perm[:, None], _GDN, (1,),
                      mode=lax.GatherScatterMode.PROMISE_IN_BOUNDS)


def _allsum(v, perms):
    for p in perms:
        v = v + _permute(v, p)
    return v


def _rsqrt(a):
    i = plsc.bitcast(a, jnp.int32)
    i = 0x5F3759DF - lax.shift_right_logical(i, 1)
    y = plsc.bitcast(i, jnp.float32)
    for _ in range(3):
        y = y * (1.5 - (0.5 * a) * y * y)
    return y


def _body(x_hbm, table_hbm, pos_hbm, gamma_hbm, beta_hbm, out_hbm,
          idx_v, rows_v, pos_v, gb_v, sem):
    wid = lax.axis_index("s") * NC + lax.axis_index("c")
    base_row = wid * ROWS_PER_W
    base_idx_row = wid * (ROWS_PER_W // IDX_W)

    pltpu.sync_copy(pos_hbm, pos_v)
    pltpu.sync_copy(gamma_hbm, gb_v.at[0])
    pltpu.sync_copy(beta_hbm, gb_v.at[1])

    g = [gb_v[0, pl.ds(i * L, L)] for i in range(EMBED_DIM // L)]
    b = [gb_v[1, pl.ds(i * L, L)] for i in range(EMBED_DIM // L)]

    lanes = lax.iota(jnp.int32, L)
    perms = [lax.bitwise_xor(lanes, jnp.int32(1 << i)) for i in range(4)]

    def chunk_body(k, carry):
        chunk_base = base_row + k * CHUNK
        pltpu.sync_copy(x_hbm.at[pl.ds(base_idx_row + k * GATHERS, GATHERS)],
                        idx_v)
        descs = [
            pltpu.async_copy(table_hbm.at[idx_v.at[j]],
                             rows_v.at[pl.ds(j * IDX_W, IDX_W)], sem)
            for j in range(GATHERS)
        ]
        for d in descs:
            d.wait()

        def row_body(r, _):
            s_pos = lax.rem(chunk_base + r, SEQ_LEN)
            h = [rows_v[r, pl.ds(i * L, L)] + pos_v[s_pos, pl.ds(i * L, L)]
                 for i in range(EMBED_DIM // L)]
            tot = (h[0] + h[1]) + (h[2] + h[3])
            sq = (h[0] * h[0] + h[1] * h[1]) + (h[2] * h[2] + h[3] * h[3])
            mean = _allsum(tot, perms) * _INV_D
            var = _allsum(sq, perms) * _INV_D - mean * mean
            rstd = _rsqrt(var + _EPS)
            for i in range(EMBED_DIM // L):
                rows_v[r, pl.ds(i * L, L)] = ((h[i] - mean) * rstd) * g[i] + b[i]
            return 0

        lax.fori_loop(0, CHUNK, row_body, 0)

        pltpu.sync_copy(rows_v, out_hbm.at[pl.ds(chunk_base, CHUNK)])
        return carry

    lax.fori_loop(0, N_CHUNKS, chunk_body, 0)


@jax.jit
def _run(x_flat2d, table, pos2d, gamma, beta):
    mesh = plsc.VectorSubcoreMesh(core_axis_name="c", subcore_axis_name="s",
                                  num_cores=NC, num_subcores=NS)
    f = pl.kernel(
        _body,
        out_type=jax.ShapeDtypeStruct((TOTAL_ROWS, EMBED_DIM), jnp.float32),
        mesh=mesh,
        compiler_params=pltpu.CompilerParams(needs_layout_passes=False,
                                             use_tc_tiling_on_sc=False),
        scratch_types=[
            pltpu.VMEM((GATHERS, IDX_W), jnp.int32),
            pltpu.VMEM((CHUNK, EMBED_DIM), jnp.float32),
            pltpu.VMEM((SEQ_LEN, EMBED_DIM), jnp.float32),
            pltpu.VMEM((2, EMBED_DIM), jnp.float32),
            pltpu.SemaphoreType.DMA,
        ],
    )
    return f(x_flat2d, table, pos2d, gamma, beta)


def kernel(x, token_table, pos_embed, gamma, beta):
    x_flat2d = x.reshape(TOTAL_ROWS // IDX_W, IDX_W)
    pos2d = pos_embed.reshape(MAX_SEQ_LEN, EMBED_DIM)[:SEQ_LEN]
    out = _run(x_flat2d, token_table, pos2d, gamma, beta)
    return out.reshape(BATCH, SEQ_LEN, EMBED_DIM)


# speedup vs baseline: 1.7023x; 1.7023x over previous
import jax
import jax.numpy as jnp
from jax import lax
from jax.experimental import pallas as pl
from jax.experimental.pallas import tpu as pltpu
from jax.experimental.pallas import tpu_sc as plsc

VOCAB = 1000000
EMBED_DIM = 64
MAX_SEQ_LEN = 200
SEQ_LEN = 200
BATCH = 4096

NC = 2
NS = 16
NW = NC * NS
L = 16
NV = EMBED_DIM // L

TOTAL_ROWS = BATCH * SEQ_LEN
ROWS_PER_W = TOTAL_ROWS // NW
CHUNK = 256
N_CHUNKS = ROWS_PER_W // CHUNK
IDX_W = 128
GATHERS = CHUNK // IDX_W
IDX_ROWS = ROWS_PER_W // IDX_W
S_STEP = CHUNK % SEQ_LEN

_EPS = 1e-5
_INV_D = 1.0 / EMBED_DIM

_GDN = lax.GatherDimensionNumbers(
    offset_dims=(), collapsed_slice_dims=(0,), start_index_map=(0,))


def _permute(v, perm):
    return lax.gather(v, perm[:, None], _GDN, (1,),
                      mode=lax.GatherScatterMode.PROMISE_IN_BOUNDS)


def _allsum(v, perms):
    for p in perms:
        v = v + _permute(v, p)
    return v


def _rsqrt(a):
    i = plsc.bitcast(a, jnp.int32)
    i = 0x5F3759DF - lax.shift_right_logical(i, 1)
    y = plsc.bitcast(i, jnp.float32)
    half_a = 0.5 * a
    for _ in range(2):
        y = y * (1.5 - half_a * y * y)
    return y


def _body(x_hbm, table_hbm, pos_hbm, gamma_hbm, beta_hbm, out_hbm,
          idx_v, ib0, ib1, ob0, ob1, pos_v, gb_v,
          gsem0, gsem1, wsem0, wsem1):
    wid = lax.axis_index("s") * NC + lax.axis_index("c")
    base_row = wid * ROWS_PER_W

    pltpu.sync_copy(x_hbm.at[pl.ds(wid * IDX_ROWS, IDX_ROWS)], idx_v)
    pltpu.sync_copy(pos_hbm, pos_v)
    pltpu.sync_copy(gamma_hbm, gb_v.at[0])
    pltpu.sync_copy(beta_hbm, gb_v.at[1])

    g = [gb_v[0, pl.ds(i * L, L)] for i in range(NV)]
    b = [gb_v[1, pl.ds(i * L, L)] for i in range(NV)]
    lanes = lax.iota(jnp.int32, L)
    perms = [lax.bitwise_xor(lanes, jnp.int32(1 << i)) for i in range(4)]

    def fire_gather(k, ibuf, gsem):
        for j in range(GATHERS):
            pltpu.async_copy(table_hbm.at[idx_v.at[k * GATHERS + j]],
                             ibuf.at[pl.ds(j * IDX_W, IDX_W)], gsem)

    def wait_gather(ibuf, gsem):
        pltpu.make_async_copy(out_hbm.at[pl.ds(0, CHUNK)], ibuf, gsem).wait()

    def fire_write(k, obuf, wsem):
        pltpu.async_copy(obuf, out_hbm.at[pl.ds(base_row + k * CHUNK, CHUNK)],
                         wsem)

    def wait_write(obuf, wsem):
        pltpu.make_async_copy(obuf, out_hbm.at[pl.ds(0, CHUNK)], wsem).wait()

    def compute(ibuf, obuf, s_base):
        def row(r):
            s = s_base + r
            s = jnp.where(s >= SEQ_LEN, s - SEQ_LEN, s)
            s = jnp.where(s >= SEQ_LEN, s - SEQ_LEN, s)
            h = [ibuf[r, pl.ds(i * L, L)] + pos_v[s, pl.ds(i * L, L)]
                 for i in range(NV)]
            tot = (h[0] + h[1]) + (h[2] + h[3])
            sq = (h[0] * h[0] + h[1] * h[1]) + (h[2] * h[2] + h[3] * h[3])
            mean = _allsum(tot, perms) * _INV_D
            var = _allsum(sq, perms) * _INV_D - mean * mean
            rstd = _rsqrt(var + _EPS)
            off = mean * rstd
            for i in range(NV):
                obuf[r, pl.ds(i * L, L)] = (h[i] * rstd - off) * g[i] + b[i]

        plsc.parallel_loop(0, CHUNK, unroll=4)(row)

    def wrap(s):
        return jnp.where(s >= SEQ_LEN, s - SEQ_LEN, s)

    fire_gather(0, ib0, gsem0)
    fire_gather(1, ib1, gsem1)

    wait_gather(ib0, gsem0)
    compute(ib0, ob0, jnp.int32(0))
    fire_write(0, ob0, wsem0)
    fire_gather(2, ib0, gsem0)
    wait_gather(ib1, gsem1)
    compute(ib1, ob1, jnp.int32(S_STEP))
    fire_write(1, ob1, wsem1)
    fire_gather(3, ib1, gsem1)

    def pair_body(j, s_base):
        k0 = 2 * j + 2
        wait_gather(ib0, gsem0)
        wait_write(ob0, wsem0)
        compute(ib0, ob0, s_base)
        fire_write(k0, ob0, wsem0)
        fire_gather(k0 + 2, ib0, gsem0)
        s_base = wrap(s_base + S_STEP)
        wait_gather(ib1, gsem1)
        wait_write(ob1, wsem1)
        compute(ib1, ob1, s_base)
        fire_write(k0 + 1, ob1, wsem1)
        fire_gather(k0 + 3, ib1, gsem1)
        return wrap(s_base + S_STEP)

    s_base = lax.fori_loop(0, (N_CHUNKS - 4) // 2, pair_body,
                           wrap(jnp.int32(2 * S_STEP)))

    wait_gather(ib0, gsem0)
    wait_write(ob0, wsem0)
    compute(ib0, ob0, s_base)
    fire_write(N_CHUNKS - 2, ob0, wsem0)
    s_base = wrap(s_base + S_STEP)
    wait_gather(ib1, gsem1)
    wait_write(ob1, wsem1)
    compute(ib1, ob1, s_base)
    fire_write(N_CHUNKS - 1, ob1, wsem1)

    wait_write(ob0, wsem0)
    wait_write(ob1, wsem1)


@jax.jit
def _run(x_flat2d, table, pos2d, gamma, beta):
    mesh = plsc.VectorSubcoreMesh(core_axis_name="c", subcore_axis_name="s",
                                  num_cores=NC, num_subcores=NS)
    f = pl.kernel(
        _body,
        out_type=jax.ShapeDtypeStruct((TOTAL_ROWS, EMBED_DIM), jnp.float32),
        mesh=mesh,
        compiler_params=pltpu.CompilerParams(needs_layout_passes=False,
                                             use_tc_tiling_on_sc=False),
        scratch_types=[
            pltpu.VMEM((IDX_ROWS, IDX_W), jnp.int32),
            pltpu.VMEM((CHUNK, EMBED_DIM), jnp.float32),
            pltpu.VMEM((CHUNK, EMBED_DIM), jnp.float32),
            pltpu.VMEM((CHUNK, EMBED_DIM), jnp.float32),
            pltpu.VMEM((CHUNK, EMBED_DIM), jnp.float32),
            pltpu.VMEM((SEQ_LEN, EMBED_DIM), jnp.float32),
            pltpu.VMEM((2, EMBED_DIM), jnp.float32),
            pltpu.SemaphoreType.DMA,
            pltpu.SemaphoreType.DMA,
            pltpu.SemaphoreType.DMA,
            pltpu.SemaphoreType.DMA,
        ],
    )
    return f(x_flat2d, table, pos2d, gamma, beta)


def kernel(x, token_table, pos_embed, gamma, beta):
    x_flat2d = x.reshape(TOTAL_ROWS // IDX_W, IDX_W)
    pos2d = pos_embed.reshape(MAX_SEQ_LEN, EMBED_DIM)[:SEQ_LEN]
    out = _run(x_flat2d, token_table, pos2d, gamma, beta)
    return out.reshape(BATCH, SEQ_LEN, EMBED_DIM)
